# Initial kernel scaffold; baseline (speedup 1.0000x reference)
import jax
import jax.numpy as jnp
from jax.experimental import pallas as pl


def kernel(points, W1, b1, W2, b2, W3, b3):
    raise NotImplementedError("write your pallas kernel here")


# speedup vs baseline: 21.5405x; 21.5405x over previous
import functools

import jax
import jax.numpy as jnp
from jax import lax
from jax.experimental import pallas as pl
from jax.experimental.pallas import tpu as pltpu

N = 2048
K_NEIGH = 32
TOPK_BITS = 31


def _tc_body(p_ref, pt_ref, w1_ref, b1_ref, w2_ref, b2_ref, w3_ref, b3_ref,
             out_ref):
    p = p_ref[0]
    pt = pt_ref[0]

    d = jnp.zeros((N, N), jnp.float32)
    for c in range(3):
        diff = p[:, c:c + 1] - pt[c:c + 1, :]
        d = d + diff * diff

    res = jnp.zeros((N, 1), jnp.int32)
    for bit in range(TOPK_BITS - 1, -1, -1):
        trial = res | (1 << bit)
        trial_f = lax.bitcast_convert_type(trial, jnp.float32)
        cnt = jnp.sum((d < trial_f).astype(jnp.float32), axis=1,
                      keepdims=True)
        res = jnp.where(cnt < float(K_NEIGH), trial, res)
    t_f = lax.bitcast_convert_type(res, jnp.float32)

    mask = d <= t_f
    cnt = jnp.sum(mask.astype(jnp.float32), axis=1, keepdims=True)

    g = jax.nn.relu(lax.dot(p, w1_ref[...], preferred_element_type=jnp.float32)
                    + b1_ref[...])
    g = jax.nn.relu(lax.dot(g, w2_ref[...], preferred_element_type=jnp.float32)
                    + b2_ref[...])
    g = jax.nn.relu(lax.dot(g, w3_ref[...], preferred_element_type=jnp.float32)
                    + b3_ref[...])

    agg = lax.dot(mask.astype(jnp.float32), g,
                  preferred_element_type=jnp.float32)
    out_ref[0] = agg / cnt


@jax.jit
def kernel(points, W1, b1, W2, b2, W3, b3):
    B = points.shape[0]
    pointsT = jnp.swapaxes(points, 1, 2)
    b1r, b2r, b3r = (b.reshape(1, -1) for b in (b1, b2, b3))

    out = pl.pallas_call(
        _tc_body,
        grid=(B,),
        in_specs=[
            pl.BlockSpec((1, N, 3), lambda b: (b, 0, 0)),
            pl.BlockSpec((1, 3, N), lambda b: (b, 0, 0)),
            pl.BlockSpec(W1.shape, lambda b: (0, 0)),
            pl.BlockSpec((1, b1.shape[0]), lambda b: (0, 0)),
            pl.BlockSpec(W2.shape, lambda b: (0, 0)),
            pl.BlockSpec((1, b2.shape[0]), lambda b: (0, 0)),
            pl.BlockSpec(W3.shape, lambda b: (0, 0)),
            pl.BlockSpec((1, b3.shape[0]), lambda b: (0, 0)),
        ],
        out_specs=pl.BlockSpec((1, N, 256), lambda b: (b, 0, 0)),
        out_shape=jax.ShapeDtypeStruct((B, N, 256), jnp.float32),
    )(points, pointsT, W1, b1r, W2, b2r, W3, b3r)
    return out
